# baseline (speedup 1.0000x reference)
import functools
import jax
import jax.numpy as jnp
from jax.experimental import pallas as pl

_DIM = 128
_BASE = 10000.0
_SEQ_BLK = 128


@functools.lru_cache(maxsize=None)
def _tables(sq):
    freqs = 1.0 / (_BASE ** (jnp.arange(0, _DIM, 2)[: _DIM // 2].astype(jnp.float32) / _DIM))
    t = jnp.arange(sq).astype(jnp.float32)
    f = jnp.outer(t, freqs)
    cos = jnp.cos(f)
    sin = jnp.sin(f)
    c_full = jnp.repeat(cos, 2, axis=1)
    s_full = jnp.stack([-sin, sin], axis=-1).reshape(sq, _DIM)
    i = jnp.arange(_DIM)
    perm = (i[:, None] ^ 1) == i[None, :]
    p = perm.astype(jnp.float32)
    return (c_full.reshape(sq, 1, _DIM), s_full.reshape(sq, 1, _DIM), p)


def _rope_body(c_ref, s_ref, p_ref, q_ref, k_ref, qo_ref, ko_ref):
    c = c_ref[...]
    s = s_ref[...]
    p = p_ref[...]
    blk, fl, hh = q_ref.shape
    for x_ref, o_ref in ((q_ref, qo_ref), (k_ref, ko_ref)):
        x = x_ref[...]
        sw = jnp.dot(x.reshape(blk * fl, hh), p,
                     preferred_element_type=jnp.float32).reshape(blk, fl, hh)
        o_ref[...] = x * c + sw * s


def kernel(query, key):
    sq, bsz, nh, hh = query.shape
    c_t, s_t, p_t = _tables(sq)
    fl = bsz * nh
    q3 = query.reshape(sq, fl, hh)
    k3 = key.reshape(sq, fl, hh)

    blk = _SEQ_BLK if sq % _SEQ_BLK == 0 else sq
    grid = (sq // blk,)
    tab_spec = pl.BlockSpec((blk, 1, hh), lambda i: (i, 0, 0))
    p_spec = pl.BlockSpec((hh, hh), lambda i: (0, 0))
    dat_spec = pl.BlockSpec((blk, fl, hh), lambda i: (i, 0, 0))

    qo, ko = pl.pallas_call(
        _rope_body,
        grid=grid,
        in_specs=[tab_spec, tab_spec, p_spec, dat_spec, dat_spec],
        out_specs=[dat_spec, dat_spec],
        out_shape=[
            jax.ShapeDtypeStruct((sq, fl, hh), query.dtype),
            jax.ShapeDtypeStruct((sq, fl, hh), key.dtype),
        ],
    )(c_t, s_t, p_t, q3, k3)
    return qo.reshape(query.shape), ko.reshape(key.shape)


# speedup vs baseline: 1.2591x; 1.2591x over previous
import functools
import jax
import jax.numpy as jnp
import numpy as np
from jax.experimental import pallas as pl

_DIM = 128
_BASE = 10000.0
_SEQ_BLK = 256


@functools.lru_cache(maxsize=None)
def _tables(sq):
    freqs = 1.0 / (_BASE ** (np.arange(0, _DIM, 2)[: _DIM // 2].astype(np.float32) / _DIM))
    t = np.arange(sq, dtype=np.float32)
    f = np.outer(t, freqs)
    cos = np.cos(f).astype(np.float32)
    sin = np.sin(f).astype(np.float32)
    c_full = np.repeat(cos, 2, axis=1)
    s_full = np.stack([-sin, sin], axis=-1).reshape(sq, _DIM)
    i = np.arange(_DIM)
    perm = (i[:, None] ^ 1) == i[None, :]
    p = perm.astype(np.float32)
    return (c_full.reshape(sq, 1, _DIM), s_full.reshape(sq, 1, _DIM), p)


def _rope_body(c_ref, s_ref, p_ref, q_ref, k_ref, qo_ref, ko_ref):
    c = c_ref[...]
    s = s_ref[...]
    p = p_ref[...]
    blk, fl, hh = q_ref.shape
    for x_ref, o_ref in ((q_ref, qo_ref), (k_ref, ko_ref)):
        x = x_ref[...]
        sw = jnp.dot(x.reshape(blk * fl, hh), p,
                     preferred_element_type=jnp.float32).reshape(blk, fl, hh)
        o_ref[...] = x * c + sw * s


def kernel(query, key):
    sq, bsz, nh, hh = query.shape
    c_t, s_t, p_t = _tables(sq)
    fl = bsz * nh
    q3 = query.reshape(sq, fl, hh)
    k3 = key.reshape(sq, fl, hh)

    blk = _SEQ_BLK if sq % _SEQ_BLK == 0 else sq
    grid = (sq // blk,)
    tab_spec = pl.BlockSpec((blk, 1, hh), lambda i: (i, 0, 0))
    p_spec = pl.BlockSpec((hh, hh), lambda i: (0, 0))
    dat_spec = pl.BlockSpec((blk, fl, hh), lambda i: (i, 0, 0))

    qo, ko = pl.pallas_call(
        _rope_body,
        grid=grid,
        in_specs=[tab_spec, tab_spec, p_spec, dat_spec, dat_spec],
        out_specs=[dat_spec, dat_spec],
        out_shape=[
            jax.ShapeDtypeStruct((sq, fl, hh), query.dtype),
            jax.ShapeDtypeStruct((sq, fl, hh), key.dtype),
        ],
    )(c_t, s_t, p_t, q3, k3)
    return qo.reshape(query.shape), ko.reshape(key.shape)
